# speedup vs baseline: 5.5256x; 5.5256x over previous
import functools

import jax
import jax.numpy as jnp
from jax import lax
from jax.experimental import pallas as pl
from jax.experimental.pallas import tpu as pltpu
from jax.experimental.pallas import tpu_sc as plsc

_N = 10000
_E = 320000
_D = 128
_NC = 2
_NS = 16
_NW = _NC * _NS
_C = 128
_CH = 80
_EW = _C * _CH
_EP = _EW * _NW
_P = 10240
_RT = _P // _NS
_ZR = 128

_mesh = plsc.VectorSubcoreMesh(core_axis_name="c", subcore_axis_name="s")


def _deg_body(src_hbm, dst_hbm, const_hbm, out_hbm,
              src_v, dst_v, ones_v, stage_v, dout_sh, din_sh):
    cid = lax.axis_index("c")
    sid = lax.axis_index("s")
    wid = sid * _NC + cid
    pltpu.sync_copy(const_hbm.at[1], ones_v)
    pltpu.sync_copy(const_hbm.at[0], stage_v)
    r0 = sid * _RT
    for i in range(_RT // _ZR):
        pltpu.sync_copy(stage_v, dout_sh.at[pl.ds(r0 + i * _ZR, _ZR)])
        pltpu.sync_copy(stage_v, din_sh.at[pl.ds(r0 + i * _ZR, _ZR)])
    plsc.subcore_barrier()
    base = wid * _EW

    def body(i, carry):
        off = base + i * _C
        pltpu.sync_copy(src_hbm.at[pl.ds(off, _C)], src_v)
        pltpu.sync_copy(dst_hbm.at[pl.ds(off, _C)], dst_v)
        pltpu.sync_copy(ones_v, dout_sh.at[src_v], add=True)
        pltpu.sync_copy(ones_v, din_sh.at[dst_v], add=True)
        return carry

    lax.fori_loop(0, _CH, body, 0)
    plsc.subcore_barrier()
    for i in range(_RT // _ZR):
        r = r0 + i * _ZR
        pltpu.sync_copy(dout_sh.at[pl.ds(r, _ZR)], stage_v)
        pltpu.sync_copy(stage_v, out_hbm.at[cid, 0, pl.ds(r, _ZR)])
        pltpu.sync_copy(din_sh.at[pl.ds(r, _ZR)], stage_v)
        pltpu.sync_copy(stage_v, out_hbm.at[cid, 1, pl.ds(r, _ZR)])


_deg_kernel = pl.kernel(
    _deg_body,
    out_type=jax.ShapeDtypeStruct((_NC, 2, _P, 16), jnp.float32),
    mesh=_mesh,
    scratch_types=[
        pltpu.VMEM((_C,), jnp.int32),
        pltpu.VMEM((_C,), jnp.int32),
        pltpu.VMEM((_C, 16), jnp.float32),
        pltpu.VMEM((_ZR, 16), jnp.float32),
        pltpu.VMEM_SHARED((_P, 16), jnp.float32),
        pltpu.VMEM_SHARED((_P, 16), jnp.float32),
    ],
)


def _agg_body(xp_hbm, src_hbm, dst_hbm, zblk_hbm, out_hbm,
              src_v, dst_v, rows_v, zero_v, acc_sh, sem):
    cid = lax.axis_index("c")
    sid = lax.axis_index("s")
    wid = sid * _NC + cid
    pltpu.sync_copy(zblk_hbm, zero_v)
    r0 = sid * _RT
    for i in range(_RT // _ZR):
        pltpu.sync_copy(zero_v, acc_sh.at[pl.ds(r0 + i * _ZR, _ZR)])
    plsc.subcore_barrier()
    base = wid * _EW

    def body(i, carry):
        off = base + i * _C
        pltpu.sync_copy(src_hbm.at[pl.ds(off, _C)], src_v)
        pltpu.async_copy(xp_hbm.at[src_v], rows_v, sem).wait()
        pltpu.sync_copy(dst_hbm.at[pl.ds(off, _C)], dst_v)
        pltpu.sync_copy(rows_v, acc_sh.at[dst_v], add=True)
        return carry

    lax.fori_loop(0, _CH, body, 0)
    plsc.subcore_barrier()
    for i in range(_RT // _ZR):
        r = r0 + i * _ZR
        pltpu.sync_copy(acc_sh.at[pl.ds(r, _ZR)], rows_v)
        pltpu.sync_copy(rows_v, out_hbm.at[cid, pl.ds(r, _ZR)])


_agg_kernel = pl.kernel(
    _agg_body,
    out_type=jax.ShapeDtypeStruct((_NC, _P, _D), jnp.float32),
    mesh=_mesh,
    scratch_types=[
        pltpu.VMEM((_C,), jnp.int32),
        pltpu.VMEM((_C,), jnp.int32),
        pltpu.VMEM((_C, _D), jnp.float32),
        pltpu.VMEM((_ZR, _D), jnp.float32),
        pltpu.VMEM_SHARED((_P, _D), jnp.float32),
        pltpu.SemaphoreType.DMA,
    ],
)


def _prep_body(dp_ref, x_ref, a_ref, b_ref, xp_ref):
    dout = dp_ref[0, 0] + dp_ref[1, 0]
    din = dp_ref[0, 1] + dp_ref[1, 1]
    a = lax.rsqrt(dout[:, :1] + 1.0)
    b = lax.rsqrt(din[:, :1] + 1.0)
    a_ref[...] = a
    b_ref[...] = b
    xp_ref[:_N] = x_ref[...] * a[:_N]
    xp_ref[_N:] = jnp.zeros((_P - _N, _D), jnp.float32)


def _layer_body(last, h_ref, s_ref, a_ref, b_ref, w_ref, bias_ref,
                g_ref, be_ref, *outs):
    h = h_ref[...]
    a = a_ref[...]
    xp = h * a
    agg = b_ref[...] * (s_ref[0] + s_ref[1] + xp)
    t = jnp.dot(agg, w_ref[...], preferred_element_type=jnp.float32)
    t = t + bias_ref[...]
    tv = t[:_N]
    mean = jnp.mean(tv, axis=0, keepdims=True)
    cen = tv - mean
    var = jnp.mean(cen * cen, axis=0, keepdims=True)
    hn = cen * lax.rsqrt(var + 1e-5) * g_ref[...] + be_ref[...]
    hn = jnp.maximum(hn, 0.0) + h[:_N]
    zpad = jnp.zeros((_P - _N, _D), jnp.float32)
    outs[0][:_N] = hn
    outs[0][_N:] = zpad
    if not last:
        outs[1][:_N] = hn * a[:_N]
        outs[1][_N:] = zpad


_prep_call = pl.pallas_call(
    _prep_body,
    out_shape=(
        jax.ShapeDtypeStruct((_P, 1), jnp.float32),
        jax.ShapeDtypeStruct((_P, 1), jnp.float32),
        jax.ShapeDtypeStruct((_P, _D), jnp.float32),
    ),
)

_layer_mid = pl.pallas_call(
    functools.partial(_layer_body, False),
    out_shape=(
        jax.ShapeDtypeStruct((_P, _D), jnp.float32),
        jax.ShapeDtypeStruct((_P, _D), jnp.float32),
    ),
)

_layer_last = pl.pallas_call(
    functools.partial(_layer_body, True),
    out_shape=jax.ShapeDtypeStruct((_P, _D), jnp.float32),
)


def kernel(features, edge_index, W1, b1, gamma1, beta1,
           W2, b2, gamma2, beta2, W3, b3, gamma3, beta3):
    src = edge_index[0].astype(jnp.int32)
    dst = edge_index[1].astype(jnp.int32)
    pad = jnp.full((_EP - _E,), _N, jnp.int32)
    srcp = jnp.concatenate([src, pad])
    dstp = jnp.concatenate([dst, pad])
    consts = jnp.stack([jnp.zeros((_C, 16), jnp.float32),
                        jnp.ones((_C, 16), jnp.float32)])
    zblk = jnp.zeros((_ZR, _D), jnp.float32)

    dp = _deg_kernel(srcp, dstp, consts)
    a, b, xp = _prep_call(dp, features)
    h = jnp.concatenate([features, jnp.zeros((_P - _N, _D), jnp.float32)])

    params = ((W1, b1, gamma1, beta1), (W2, b2, gamma2, beta2),
              (W3, b3, gamma3, beta3))
    for i, (W, bias, gamma, beta) in enumerate(params):
        S = _agg_kernel(xp, srcp, dstp, zblk)
        args = (h, S, a, b, W, bias.reshape(1, _D),
                gamma.reshape(1, _D), beta.reshape(1, _D))
        if i < 2:
            h, xp = _layer_mid(*args)
        else:
            h = _layer_last(*args)
    return h[:_N]
